# Initial kernel scaffold; baseline (speedup 1.0000x reference)
import jax
import jax.numpy as jnp
from jax.experimental import pallas as pl


def kernel(x, star_features, star_labels, d1_features, d1_labels, d2_features, d2_labels, unique1, unique2, W1, b1, W2, b2):
    raise NotImplementedError("write your pallas kernel here")


# speedup vs baseline: 3.8959x; 3.8959x over previous
import jax
import jax.numpy as jnp
from jax.experimental import pallas as pl
from jax.experimental.pallas import tpu as pltpu

N, B, D, LD, C = 8192, 128, 128, 32, 64
BLK = 1024
NB = N // BLK
NCH = 6
CD = D * NCH
CLD = LD * NCH
HI = jax.lax.Precision.HIGHEST


def _chunks(v):
    parts = [((v >> s) & 63).astype(jnp.bfloat16) for s in (0, 6, 12, 18, 24, 30)]
    return jnp.concatenate(parts, axis=-1)


def _bits(f):
    return jax.lax.bitcast_convert_type(f, jnp.int32)


def _dot_t(a, b, prec=None):
    return jax.lax.dot_general(a, b, (((1,), (1,)), ((), ())),
                               precision=prec, preferred_element_type=jnp.float32)


def _match_gather_kernel(x_ref, sf_ref, d1_ref, d2_ref, x1_ref, x2_ref,
                         xc_ref, found_ref):
    j = pl.program_id(0)

    @pl.when(j == 0)
    def _init():
        xc_ref[...] = _chunks(_bits(x_ref[...]))
        found_ref[...] = jnp.zeros_like(found_ref)
        x1_ref[...] = jnp.zeros_like(x1_ref)
        x2_ref[...] = jnp.zeros_like(x2_ref)

    sfc = _chunks(_bits(sf_ref[...]))
    xc = xc_ref[...]
    g = _dot_t(xc, sfc)
    nx = jnp.sum(xc.astype(jnp.float32) ** 2, axis=1)
    nf = jnp.sum(sfc.astype(jnp.float32) ** 2, axis=1)
    m2 = nx[:, None] + nf[None, :] - 2.0 * g
    il = jax.lax.broadcasted_iota(jnp.int32, (B, BLK), 1)
    lidx = jnp.min(jnp.where(m2 == 0.0, il, BLK), axis=1)
    take = (lidx < BLK) & (found_ref[0, :] == 0.0)
    oh = ((il == lidx[:, None]) & take[:, None]).astype(jnp.float32)
    x1_ref[...] += jax.lax.dot_general(oh, d1_ref[...], (((1,), (0,)), ((), ())),
                                       precision=HI,
                                       preferred_element_type=jnp.float32)
    x2_ref[...] += jax.lax.dot_general(oh, d2_ref[...], (((1,), (0,)), ((), ())),
                                       precision=HI,
                                       preferred_element_type=jnp.float32)
    found_ref[0, :] += take.astype(jnp.float32)


def _main_kernel(x1_ref, x2_ref, w1_ref, b1_ref, w2_ref, b2_ref, u1_ref, u2_ref,
                 d1f_ref, d1l_ref, d2f_ref, d2l_ref, slb_ref, out_ref,
                 u1c_ref, u2c_ref, qc1_ref, qc2_ref, nx1_ref, nx2_ref,
                 num1_ref, den1_ref, num2_ref, den2_ref):
    j = pl.program_id(0)
    sides = (
        (x1_ref, w1_ref, b1_ref, u1_ref, u1c_ref, qc1_ref, nx1_ref,
         d1f_ref, d1l_ref, num1_ref, den1_ref),
        (x2_ref, w2_ref, b2_ref, u2_ref, u2c_ref, qc2_ref, nx2_ref,
         d2f_ref, d2l_ref, num2_ref, den2_ref),
    )

    @pl.when(j == 0)
    def _init():
        for (x_ref, w_ref, b_ref, u_ref, uc_ref, qc_ref, nx_ref,
             _df, _dl, num_ref, den_ref) in sides:
            u = u_ref[...]
            uc = _chunks(_bits(u))
            uc_ref[...] = uc
            nu = jnp.sum(uc.astype(jnp.float32) ** 2, axis=1)
            m2u = nu[:, None] + nu[None, :] - 2.0 * _dot_t(uc, uc)
            ic = jax.lax.broadcasted_iota(jnp.int32, (C, C), 1)
            ucanon = jnp.min(jnp.where(m2u == 0.0, ic, C), axis=1).astype(jnp.float32)
            xg = x_ref[...]
            y = jax.lax.dot_general(xg, w_ref[...], (((1,), (0,)), ((), ())),
                                    precision=HI,
                                    preferred_element_type=jnp.float32)
            y = y + b_ref[0, :][None, :]
            ny = jnp.sum(y * y, axis=1)
            nuf = jnp.sum(u * u, axis=1)
            dq = ny[:, None] + nuf[None, :] - 2.0 * _dot_t(y, u, HI)
            mn = jnp.min(dq, axis=1, keepdims=True)
            cb = jax.lax.broadcasted_iota(jnp.int32, (B, C), 1)
            cidx = jnp.min(jnp.where(dq == mn, cb, C), axis=1)
            qc_ref[0, :] = jnp.sum(
                jnp.where(cb == cidx[:, None], ucanon[None, :], 0.0), axis=1)
            nx_ref[0, :] = jnp.sum(xg * xg, axis=1)
            num_ref[...] = jnp.zeros_like(num_ref)
            den_ref[...] = jnp.zeros_like(den_ref)

    slb = slb_ref[...]
    for (x_ref, _w, _b, _u, uc_ref, qc_ref, nx_ref,
         df_ref, dl_ref, num_ref, den_ref) in sides:
        f = df_ref[...]
        uc = uc_ref[...]
        lc = _chunks(_bits(dl_ref[...]))
        nl = jnp.sum(lc.astype(jnp.float32) ** 2, axis=1)
        nu = jnp.sum(uc.astype(jnp.float32) ** 2, axis=1)
        m2l = nl[:, None] + nu[None, :] - 2.0 * _dot_t(lc, uc)
        icc = jax.lax.broadcasted_iota(jnp.int32, (BLK, C), 1)
        rowc = jnp.min(jnp.where(m2l == 0.0, icc, C), axis=1).astype(jnp.float32)
        mask = (qc_ref[0, :][:, None] == rowc[None, :]).astype(jnp.float32)
        g = _dot_t(x_ref[...], f, HI)
        nf = jnp.sum(f * f, axis=1)
        sq = nx_ref[0, :][:, None] + nf[None, :] - 2.0 * g
        expo = jnp.exp(-sq) * mask
        num_ref[...] += jax.lax.dot_general(expo, slb, (((1,), (0,)), ((), ())),
                                            precision=HI,
                                            preferred_element_type=jnp.float32)
        den_ref[0, :] += jnp.sum(expo, axis=1)

    @pl.when(j == NB - 1)
    def _fin():
        out_ref[...] = 0.5 * (num1_ref[...] / den1_ref[0, :][:, None]
                              + num2_ref[...] / den2_ref[0, :][:, None])


def kernel(x, star_features, star_labels, d1_features, d1_labels,
           d2_features, d2_labels, unique1, unique2, W1, b1, W2, b2):
    f32 = jnp.float32
    x1, x2 = pl.pallas_call(
        _match_gather_kernel,
        grid=(NB,),
        in_specs=[
            pl.BlockSpec((B, D), lambda j: (0, 0)),
            pl.BlockSpec((BLK, D), lambda j: (j, 0)),
            pl.BlockSpec((BLK, D), lambda j: (j, 0)),
            pl.BlockSpec((BLK, D), lambda j: (j, 0)),
        ],
        out_specs=[
            pl.BlockSpec((B, D), lambda j: (0, 0)),
            pl.BlockSpec((B, D), lambda j: (0, 0)),
        ],
        out_shape=[
            jax.ShapeDtypeStruct((B, D), f32),
            jax.ShapeDtypeStruct((B, D), f32),
        ],
        scratch_shapes=[
            pltpu.VMEM((B, CD), jnp.bfloat16),
            pltpu.VMEM((1, B), f32),
        ],
    )(x, star_features, d1_features, d2_features)

    s = pl.pallas_call(
        _main_kernel,
        grid=(NB,),
        in_specs=[
            pl.BlockSpec((B, D), lambda j: (0, 0)),
            pl.BlockSpec((B, D), lambda j: (0, 0)),
            pl.BlockSpec((D, LD), lambda j: (0, 0)),
            pl.BlockSpec((1, LD), lambda j: (0, 0)),
            pl.BlockSpec((D, LD), lambda j: (0, 0)),
            pl.BlockSpec((1, LD), lambda j: (0, 0)),
            pl.BlockSpec((C, LD), lambda j: (0, 0)),
            pl.BlockSpec((C, LD), lambda j: (0, 0)),
            pl.BlockSpec((BLK, D), lambda j: (j, 0)),
            pl.BlockSpec((BLK, LD), lambda j: (j, 0)),
            pl.BlockSpec((BLK, D), lambda j: (j, 0)),
            pl.BlockSpec((BLK, LD), lambda j: (j, 0)),
            pl.BlockSpec((BLK, LD), lambda j: (j, 0)),
        ],
        out_specs=pl.BlockSpec((B, LD), lambda j: (0, 0)),
        out_shape=jax.ShapeDtypeStruct((B, LD), f32),
        scratch_shapes=[
            pltpu.VMEM((C, CLD), jnp.bfloat16),
            pltpu.VMEM((C, CLD), jnp.bfloat16),
            pltpu.VMEM((1, B), f32),
            pltpu.VMEM((1, B), f32),
            pltpu.VMEM((1, B), f32),
            pltpu.VMEM((1, B), f32),
            pltpu.VMEM((B, LD), f32),
            pltpu.VMEM((1, B), f32),
            pltpu.VMEM((B, LD), f32),
            pltpu.VMEM((1, B), f32),
        ],
    )(x1, x2, W1, b1.reshape(1, LD), W2, b2.reshape(1, LD),
      unique1, unique2, d1_features, d1_labels, d2_features, d2_labels,
      star_labels)
    return s


# speedup vs baseline: 4.7549x; 1.2205x over previous
import jax
import jax.numpy as jnp
from jax.experimental import pallas as pl
from jax.experimental.pallas import tpu as pltpu

N, B, D, LD, C = 8192, 128, 128, 32, 64
BLK = 1024
NB = N // BLK
CD = D * 5
CLD = LD * 4
HI = jax.lax.Precision.HIGHEST


def _chunks7(v):
    parts = [((v >> s) & 127).astype(jnp.bfloat16) for s in (0, 7, 14, 21, 28)]
    return jnp.concatenate(parts, axis=-1)


def _chunks8(v):
    parts = [((v >> s) & 255).astype(jnp.bfloat16) for s in (0, 8, 16, 24)]
    return jnp.concatenate(parts, axis=-1)


def _bits(f):
    return jax.lax.bitcast_convert_type(f, jnp.int32)


def _dot_t(a, b, prec=None):
    return jax.lax.dot_general(a, b, (((1,), (1,)), ((), ())),
                               precision=prec, preferred_element_type=jnp.float32)


def _match_gather_kernel(x_ref, sf_ref, d1_ref, d2_ref, x1_ref, x2_ref,
                         xc_ref, found_ref):
    j = pl.program_id(0)

    @pl.when(j == 0)
    def _init():
        xc_ref[...] = _chunks7(_bits(x_ref[...]))
        found_ref[...] = jnp.zeros_like(found_ref)
        x1_ref[...] = jnp.zeros_like(x1_ref)
        x2_ref[...] = jnp.zeros_like(x2_ref)

    sfc = _chunks7(_bits(sf_ref[...]))
    xc = xc_ref[...]
    g = _dot_t(xc, sfc)
    nx = jnp.sum(xc.astype(jnp.float32) ** 2, axis=1)
    nf = jnp.sum(sfc.astype(jnp.float32) ** 2, axis=1)
    m2 = nx[:, None] + nf[None, :] - 2.0 * g
    il = jax.lax.broadcasted_iota(jnp.int32, (B, BLK), 1)
    lidx = jnp.min(jnp.where(m2 == 0.0, il, BLK), axis=1)
    take = (lidx < BLK) & (found_ref[0, :] == 0.0)
    oh = ((il == lidx[:, None]) & take[:, None]).astype(jnp.float32)
    x1_ref[...] += jax.lax.dot_general(oh, d1_ref[...], (((1,), (0,)), ((), ())),
                                       precision=HI,
                                       preferred_element_type=jnp.float32)
    x2_ref[...] += jax.lax.dot_general(oh, d2_ref[...], (((1,), (0,)), ((), ())),
                                       precision=HI,
                                       preferred_element_type=jnp.float32)
    found_ref[0, :] += take.astype(jnp.float32)


def _main_kernel(x1_ref, x2_ref, w1_ref, b1_ref, w2_ref, b2_ref, u1_ref, u2_ref,
                 d1f_ref, d1l_ref, d2f_ref, d2l_ref, slb_ref, out_ref,
                 u1c_ref, u2c_ref, oh1_ref, oh2_ref, nx1_ref, nx2_ref,
                 num1_ref, num2_ref):
    j = pl.program_id(0)
    sides = (
        (x1_ref, w1_ref, b1_ref, u1_ref, u1c_ref, oh1_ref, nx1_ref,
         d1f_ref, d1l_ref, num1_ref),
        (x2_ref, w2_ref, b2_ref, u2_ref, u2c_ref, oh2_ref, nx2_ref,
         d2f_ref, d2l_ref, num2_ref),
    )

    @pl.when(j == 0)
    def _init():
        for (x_ref, w_ref, b_ref, u_ref, uc_ref, oh_ref, nx_ref,
             _df, _dl, num_ref) in sides:
            u = u_ref[...]
            uc_ref[...] = _chunks8(_bits(u))
            xg = x_ref[...]
            y = jax.lax.dot_general(xg, w_ref[...], (((1,), (0,)), ((), ())),
                                    precision=HI,
                                    preferred_element_type=jnp.float32)
            y = y + b_ref[0, :][None, :]
            ny = jnp.sum(y * y, axis=1)
            nuf = jnp.sum(u * u, axis=1)
            dq = ny[:, None] + nuf[None, :] - 2.0 * _dot_t(y, u, HI)
            mn = jnp.min(dq, axis=1, keepdims=True)
            cb = jax.lax.broadcasted_iota(jnp.int32, (B, C), 1)
            cidx = jnp.min(jnp.where(dq == mn, cb, C), axis=1)
            oh_ref[...] = (cb == cidx[:, None]).astype(jnp.bfloat16)
            nx_ref[0, :] = jnp.sum(xg * xg, axis=1)
            num_ref[...] = jnp.zeros_like(num_ref)

    slb = slb_ref[...]
    slb_ext = jnp.concatenate(
        [slb, jnp.ones((BLK, 1), jnp.float32)], axis=1).astype(jnp.bfloat16)
    for (x_ref, _w, _b, _u, uc_ref, oh_ref, nx_ref,
         df_ref, dl_ref, num_ref) in sides:
        f = df_ref[...]
        uc = uc_ref[...]
        lc = _chunks8(_bits(dl_ref[...]))
        nl = jnp.sum(lc.astype(jnp.float32) ** 2, axis=1)
        nu = jnp.sum(uc.astype(jnp.float32) ** 2, axis=1)
        m2l = nl[:, None] + nu[None, :] - 2.0 * _dot_t(lc, uc)
        e = (m2l == 0.0).astype(jnp.bfloat16)
        mask = _dot_t(oh_ref[...], e)
        g = _dot_t(x_ref[...], f, HI)
        nf = jnp.sum(f * f, axis=1)
        sq = nx_ref[0, :][:, None] + nf[None, :] - 2.0 * g
        expo = (jnp.exp(-sq) * mask).astype(jnp.bfloat16)
        num_ref[...] += jax.lax.dot_general(
            expo, slb_ext, (((1,), (0,)), ((), ())),
            preferred_element_type=jnp.float32)

    @pl.when(j == NB - 1)
    def _fin():
        n1 = num1_ref[...]
        n2 = num2_ref[...]
        out_ref[...] = 0.5 * (n1[:, :LD] / n1[:, LD:LD + 1]
                              + n2[:, :LD] / n2[:, LD:LD + 1])


def kernel(x, star_features, star_labels, d1_features, d1_labels,
           d2_features, d2_labels, unique1, unique2, W1, b1, W2, b2):
    f32 = jnp.float32
    x1, x2 = pl.pallas_call(
        _match_gather_kernel,
        grid=(NB,),
        in_specs=[
            pl.BlockSpec((B, D), lambda j: (0, 0)),
            pl.BlockSpec((BLK, D), lambda j: (j, 0)),
            pl.BlockSpec((BLK, D), lambda j: (j, 0)),
            pl.BlockSpec((BLK, D), lambda j: (j, 0)),
        ],
        out_specs=[
            pl.BlockSpec((B, D), lambda j: (0, 0)),
            pl.BlockSpec((B, D), lambda j: (0, 0)),
        ],
        out_shape=[
            jax.ShapeDtypeStruct((B, D), f32),
            jax.ShapeDtypeStruct((B, D), f32),
        ],
        scratch_shapes=[
            pltpu.VMEM((B, CD), jnp.bfloat16),
            pltpu.VMEM((1, B), f32),
        ],
    )(x, star_features, d1_features, d2_features)

    s = pl.pallas_call(
        _main_kernel,
        grid=(NB,),
        in_specs=[
            pl.BlockSpec((B, D), lambda j: (0, 0)),
            pl.BlockSpec((B, D), lambda j: (0, 0)),
            pl.BlockSpec((D, LD), lambda j: (0, 0)),
            pl.BlockSpec((1, LD), lambda j: (0, 0)),
            pl.BlockSpec((D, LD), lambda j: (0, 0)),
            pl.BlockSpec((1, LD), lambda j: (0, 0)),
            pl.BlockSpec((C, LD), lambda j: (0, 0)),
            pl.BlockSpec((C, LD), lambda j: (0, 0)),
            pl.BlockSpec((BLK, D), lambda j: (j, 0)),
            pl.BlockSpec((BLK, LD), lambda j: (j, 0)),
            pl.BlockSpec((BLK, D), lambda j: (j, 0)),
            pl.BlockSpec((BLK, LD), lambda j: (j, 0)),
            pl.BlockSpec((BLK, LD), lambda j: (j, 0)),
        ],
        out_specs=pl.BlockSpec((B, LD), lambda j: (0, 0)),
        out_shape=jax.ShapeDtypeStruct((B, LD), f32),
        scratch_shapes=[
            pltpu.VMEM((C, CLD), jnp.bfloat16),
            pltpu.VMEM((C, CLD), jnp.bfloat16),
            pltpu.VMEM((B, C), jnp.bfloat16),
            pltpu.VMEM((B, C), jnp.bfloat16),
            pltpu.VMEM((1, B), f32),
            pltpu.VMEM((1, B), f32),
            pltpu.VMEM((B, LD + 1), f32),
            pltpu.VMEM((B, LD + 1), f32),
        ],
    )(x1, x2, W1, b1.reshape(1, LD), W2, b2.reshape(1, LD),
      unique1, unique2, d1_features, d1_labels, d2_features, d2_labels,
      star_labels)
    return s
